# baseline (speedup 1.0000x reference)
import functools

import jax
import jax.numpy as jnp
from jax import lax
from jax.experimental import pallas as pl
from jax.experimental.pallas import tpu as pltpu
from jax.experimental.pallas import tpu_sc as plsc

NUM_CORES = 2
NUM_SUBCORES = 16
NW = NUM_CORES * NUM_SUBCORES
IDX_CHUNK = 128


def _build_gather(batch: int, depth: int):
    b_per_w = batch // NW
    n_chunks = b_per_w // IDX_CHUNK
    mesh = plsc.VectorSubcoreMesh(core_axis_name="c", subcore_axis_name="s")

    @functools.partial(
        pl.kernel,
        mesh=mesh,
        out_type=jax.ShapeDtypeStruct((batch, depth), jnp.float32),
        scratch_types=[
            pltpu.VMEM((n_chunks, IDX_CHUNK), jnp.int32),
            pltpu.VMEM((b_per_w, depth), jnp.float32),
            pltpu.SemaphoreType.DMA,
        ],
        compiler_params=pltpu.CompilerParams(use_tc_tiling_on_sc=False),
    )
    def gather_kernel(table_hbm, idx_hbm, out_hbm, idx_v, rows_v, sem):
        wid = lax.axis_index("s") * NUM_CORES + lax.axis_index("c")
        base = wid * b_per_w
        pltpu.sync_copy(idx_hbm.at[pl.ds(wid * n_chunks, n_chunks)], idx_v)
        copies = []
        for j in range(n_chunks):
            copies.append(
                pltpu.async_copy(
                    table_hbm.at[idx_v.at[j]],
                    rows_v.at[pl.ds(j * IDX_CHUNK, IDX_CHUNK)],
                    sem,
                )
            )
        for c in copies:
            c.wait()
        pltpu.sync_copy(rows_v, out_hbm.at[pl.ds(base, b_per_w)])

    return gather_kernel


def kernel(x, emb):
    batch = x.shape[0]
    depth = emb.shape[1]
    idx = x.reshape(NW * (batch // NW // IDX_CHUNK), IDX_CHUNK).astype(jnp.int32)
    gather = _build_gather(batch, depth)
    return gather(emb, idx)


# speedup vs baseline: 1.7035x; 1.7035x over previous
import functools

import jax
import jax.numpy as jnp
from jax import lax
from jax.experimental import pallas as pl
from jax.experimental.pallas import tpu as pltpu
from jax.experimental.pallas import tpu_sc as plsc

NUM_CORES = 2
NUM_SUBCORES = 16
NW = NUM_CORES * NUM_SUBCORES
CH = 64


def _build_gather(batch: int, depth: int):
    b_per_w = batch // NW
    n_chunks = b_per_w // CH
    mesh = plsc.VectorSubcoreMesh(core_axis_name="c", subcore_axis_name="s")

    @functools.partial(
        pl.kernel,
        mesh=mesh,
        out_type=jax.ShapeDtypeStruct((batch, depth), jnp.float32),
        scratch_types=[
            pltpu.SMEM((b_per_w,), jnp.int32),
            pltpu.VMEM_SHARED((NUM_SUBCORES, b_per_w), jnp.int32),
            pltpu.VMEM((b_per_w, depth), jnp.float32),
            pltpu.SemaphoreType.DMA,
        ],
    )
    def gather_kernel(table_hbm, idx_hbm, out_hbm, xs, xsh, rows, sem):
        sid = lax.axis_index("s")
        wid = sid * NUM_CORES + lax.axis_index("c")
        base = wid * b_per_w
        pltpu.sync_copy(idx_hbm.at[pl.ds(base, b_per_w)], xsh.at[sid])
        pltpu.sync_copy(xsh.at[sid], xs)

        def row_body(j, carry):
            pltpu.async_copy(table_hbm.at[xs[j]], rows.at[j], sem)
            return carry

        for c in range(n_chunks):
            lax.fori_loop(c * CH, (c + 1) * CH, row_body, None)
            pltpu.make_async_copy(
                table_hbm.at[pl.ds(0, CH)],
                rows.at[pl.ds(c * CH, CH)],
                sem,
            ).wait()
        pltpu.sync_copy(rows, out_hbm.at[pl.ds(base, b_per_w)])

    return gather_kernel


def kernel(x, emb):
    batch = x.shape[0]
    depth = emb.shape[1]
    idx = x.reshape(batch).astype(jnp.int32)
    gather = _build_gather(batch, depth)
    return gather(emb, idx)
